# baseline (speedup 1.0000x reference)
import jax
import jax.numpy as jnp
from jax import lax
from jax.experimental import pallas as pl
from jax.experimental.pallas import tpu as pltpu
from jax.experimental.pallas import tpu_sc as plsc

NC = 2
NS = 16
NW = NC * NS

BATCH = 16384
SEQ = 50
DIM = 64
BPW = BATCH // NW
CB = 256
CPS = BPW // CB
NCHUNK = SEQ * CPS
NGRP = CB // 16


def _body(xt_hbm, table_hbm, out_hbm, idx_all, rows, trows, gsem, ssem, isem):
    wid = lax.axis_index("s") * NC + lax.axis_index("c")
    col0 = wid * BPW

    pltpu.async_copy(xt_hbm.at[:, pl.ds(col0, BPW)], idx_all, isem).wait()

    lanes = lax.iota(jnp.int32, 16)

    def gather_start(c, b):
        t = c // CPS
        half = c % CPS
        src = table_hbm.at[idx_all.at[t, pl.ds(half * CB, CB)]]
        pltpu.async_copy(src, rows.at[b], gsem.at[b])

    def gather_wait(b):
        pltpu.make_async_copy(table_hbm.at[idx_all.at[0, pl.ds(0, CB)]],
                              rows.at[b], gsem.at[b]).wait()

    def store_start(c, b):
        t = c // CPS
        half = c % CPS
        dst = out_hbm.at[t, :, pl.ds(col0 + half * CB, CB)]
        pltpu.async_copy(trows.at[b], dst, ssem.at[b])

    def store_wait(b):
        pltpu.make_async_copy(trows.at[b], out_hbm.at[0, :, pl.ds(0, CB)],
                              ssem.at[b]).wait()

    def transpose(b):
        def group(g, carry):
            ridx = g * 16 + lanes
            for d in range(DIM):
                v = plsc.load_gather(rows.at[b], [ridx, jnp.full((16,), d, jnp.int32)])
                trows[b, d, pl.ds(g * 16, 16)] = v
            return carry
        lax.fori_loop(0, NGRP, group, 0)

    gather_start(0, 0)
    gather_start(1, 1)

    def step(c, carry):
        b = c % 2
        gather_wait(b)

        @pl.when(c >= 2)
        def _():
            store_wait(b)

        transpose(b)
        store_start(c, b)

        @pl.when(c + 2 < NCHUNK)
        def _():
            gather_start(c + 2, b)
        return carry

    lax.fori_loop(0, NCHUNK, step, 0)

    store_wait(0)
    store_wait(1)


@jax.jit
def _embedding_lookup(xt, table):
    mesh = plsc.VectorSubcoreMesh(core_axis_name="c", subcore_axis_name="s")
    return pl.kernel(
        _body,
        out_type=jax.ShapeDtypeStruct((SEQ, DIM, BATCH), jnp.float32),
        mesh=mesh,
        scratch_types=[
            pltpu.VMEM((SEQ, BPW), jnp.int32),
            pltpu.VMEM((2, CB, DIM), jnp.float32),
            pltpu.VMEM((2, DIM, CB), jnp.float32),
            pltpu.SemaphoreType.DMA((2,)),
            pltpu.SemaphoreType.DMA((2,)),
            pltpu.SemaphoreType.DMA,
        ],
        compiler_params=pltpu.CompilerParams(use_tc_tiling_on_sc=False,
                                             needs_layout_passes=False),
    )(xt, table)


def kernel(x, table):
    out = _embedding_lookup(x.T.astype(jnp.int32), table)
    return jnp.transpose(out, (2, 0, 1))


# speedup vs baseline: 1.2886x; 1.2886x over previous
import jax
import jax.numpy as jnp
from jax import lax
from jax.experimental import pallas as pl
from jax.experimental.pallas import tpu as pltpu
from jax.experimental.pallas import tpu_sc as plsc

NC = 2
NS = 16
NW = NC * NS

BATCH = 16384
SEQ = 50
DIM = 64
BPW = BATCH // NW
CB = 256
CPS = BPW // CB
NCHUNK = SEQ * CPS
NGRP = CB // 16


def _body(xt_hbm, table_hbm, out_hbm, idx_all, rows, trows, gsem, ssem, isem):
    wid = lax.axis_index("s") * NC + lax.axis_index("c")
    col0 = wid * BPW

    pltpu.async_copy(xt_hbm.at[:, pl.ds(col0, BPW)], idx_all, isem).wait()

    lanes = lax.iota(jnp.int32, 16)

    def gather_start(c, b):
        t = c // CPS
        half = c % CPS
        src = table_hbm.at[idx_all.at[t, pl.ds(half * CB, CB)]]
        pltpu.async_copy(src, rows.at[b], gsem.at[b])

    def gather_wait(b):
        pltpu.make_async_copy(table_hbm.at[idx_all.at[0, pl.ds(0, CB)]],
                              rows.at[b], gsem.at[b]).wait()

    def store_start(c, b):
        t = c // CPS
        half = c % CPS
        dst = out_hbm.at[t, :, pl.ds(col0 + half * CB, CB)]
        pltpu.async_copy(trows.at[b], dst, ssem.at[b])

    def store_wait(b):
        pltpu.make_async_copy(trows.at[b], out_hbm.at[0, :, pl.ds(0, CB)],
                              ssem.at[b]).wait()

    def transpose(b):
        @plsc.parallel_loop(0, NGRP, unroll=2)
        def group(g):
            ridx = g * 16 + lanes
            for d in range(DIM):
                v = plsc.load_gather(rows.at[b], [ridx, jnp.full((16,), d, jnp.int32)])
                trows[b, d, pl.ds(g * 16, 16)] = v

    gather_start(0, 0)
    gather_start(1, 1)

    def step(c, carry):
        b = c % 2
        gather_wait(b)

        @pl.when(c >= 2)
        def _():
            store_wait(b)

        transpose(b)
        store_start(c, b)

        @pl.when(c + 2 < NCHUNK)
        def _():
            gather_start(c + 2, b)
        return carry

    lax.fori_loop(0, NCHUNK, step, 0)

    store_wait(0)
    store_wait(1)


@jax.jit
def _embedding_lookup(xt, table):
    mesh = plsc.VectorSubcoreMesh(core_axis_name="c", subcore_axis_name="s")
    return pl.kernel(
        _body,
        out_type=jax.ShapeDtypeStruct((SEQ, DIM, BATCH), jnp.float32),
        mesh=mesh,
        scratch_types=[
            pltpu.VMEM((SEQ, BPW), jnp.int32),
            pltpu.VMEM((2, CB, DIM), jnp.float32),
            pltpu.VMEM((2, DIM, CB), jnp.float32),
            pltpu.SemaphoreType.DMA((2,)),
            pltpu.SemaphoreType.DMA((2,)),
            pltpu.SemaphoreType.DMA,
        ],
        compiler_params=pltpu.CompilerParams(use_tc_tiling_on_sc=False,
                                             needs_layout_passes=False),
    )(xt, table)


def kernel(x, table):
    out = _embedding_lookup(x.T.astype(jnp.int32), table)
    return jnp.transpose(out, (2, 0, 1))


# speedup vs baseline: 2.0933x; 1.6245x over previous
import jax
import jax.numpy as jnp
from jax import lax
from jax.experimental import pallas as pl
from jax.experimental.pallas import tpu as pltpu
from jax.experimental.pallas import tpu_sc as plsc

NC = 2
NS = 16
NW = NC * NS

BATCH = 16384
SEQ = 50
DIM = 64
BPW = BATCH // NW
CB = 256
CPS = BPW // CB
NCHUNK = SEQ * CPS
TCOL = CB + 1


def _body(xt_hbm, table_hbm, out_hbm, idx_all, rows, trows, gsem, ssem, isem):
    wid = lax.axis_index("s") * NC + lax.axis_index("c")
    col0 = wid * BPW

    pltpu.async_copy(xt_hbm.at[:, pl.ds(col0, BPW)], idx_all, isem).wait()

    lanes = lax.iota(jnp.int32, 16)

    def gather_start(c, b):
        t = c // CPS
        half = c % CPS
        src = table_hbm.at[idx_all.at[t, pl.ds(half * CB, CB)]]
        pltpu.async_copy(src, rows.at[b], gsem.at[b])

    def gather_wait(b):
        pltpu.make_async_copy(table_hbm.at[idx_all.at[0, pl.ds(0, CB)]],
                              rows.at[b], gsem.at[b]).wait()

    def store_start(c, b):
        t = c // CPS
        half = c % CPS
        dst = out_hbm.at[t, :, pl.ds(col0 + half * CB, CB)]
        pltpu.async_copy(trows.at[b, :, pl.ds(0, CB)], dst, ssem.at[b])

    def store_wait(b):
        pltpu.make_async_copy(trows.at[b, :, pl.ds(0, CB)],
                              out_hbm.at[0, :, pl.ds(0, CB)],
                              ssem.at[b]).wait()

    def transpose(b):
        t2 = trows.at[b]

        @plsc.parallel_loop(0, CB, unroll=2)
        def token(tk):
            cidx = tk + jnp.zeros((16,), jnp.int32)
            for dv in range(DIM // 16):
                v = rows[b, tk, pl.ds(dv * 16, 16)]
                plsc.store_scatter(t2, [dv * 16 + lanes, cidx], v)

    gather_start(0, 0)
    gather_start(1, 1)

    def step(c, carry):
        b = c % 2
        gather_wait(b)

        @pl.when(c >= 2)
        def _():
            store_wait(b)

        transpose(b)
        store_start(c, b)

        @pl.when(c + 2 < NCHUNK)
        def _():
            gather_start(c + 2, b)
        return carry

    lax.fori_loop(0, NCHUNK, step, 0)

    store_wait(0)
    store_wait(1)


@jax.jit
def _embedding_lookup(xt, table):
    mesh = plsc.VectorSubcoreMesh(core_axis_name="c", subcore_axis_name="s")
    return pl.kernel(
        _body,
        out_type=jax.ShapeDtypeStruct((SEQ, DIM, BATCH), jnp.float32),
        mesh=mesh,
        scratch_types=[
            pltpu.VMEM((SEQ, BPW), jnp.int32),
            pltpu.VMEM((2, CB, DIM), jnp.float32),
            pltpu.VMEM((2, DIM, TCOL), jnp.float32),
            pltpu.SemaphoreType.DMA((2,)),
            pltpu.SemaphoreType.DMA((2,)),
            pltpu.SemaphoreType.DMA,
        ],
        compiler_params=pltpu.CompilerParams(use_tc_tiling_on_sc=False,
                                             needs_layout_passes=False),
    )(xt, table)


def kernel(x, table):
    out = _embedding_lookup(x.T.astype(jnp.int32), table)
    return jnp.transpose(out, (2, 0, 1))
